# baseline (speedup 1.0000x reference)
import functools
import jax
import jax.numpy as jnp
from jax import lax
from jax.experimental import pallas as pl
from jax.experimental.pallas import tpu as pltpu
from jax.experimental.pallas import tpu_sc as plsc

E = 8
TILE = 128
NC, NS = 2, 16
NW = NC * NS


def _router_body(x_ref, semb_ref, wr_ref, br_ref,
                 pos0_ref, pos1_ref, w0_ref, w1_ref, te_ref):
    S = x_ref.shape[0]
    NT = te_ref.shape[0]
    logits = jnp.dot(x_ref[...] + semb_ref[...], wr_ref[...].T,
                     preferred_element_type=jnp.float32) + br_ref[...]
    ecol = lax.broadcasted_iota(jnp.int32, (S, E), 1)
    big = jnp.int32(E)
    l0 = jnp.max(logits, axis=-1, keepdims=True)
    i0 = jnp.min(jnp.where(logits == l0, ecol, big), axis=-1, keepdims=True)
    masked = jnp.where(ecol == i0, -jnp.inf, logits)
    l1 = jnp.max(masked, axis=-1, keepdims=True)
    i1 = jnp.min(jnp.where(masked == l1, ecol, big), axis=-1, keepdims=True)
    w0 = 1.0 / (1.0 + jnp.exp(l1 - l0))
    w0_ref[...] = w0
    w1_ref[...] = 1.0 - w0

    oh0 = (ecol == i0).astype(jnp.float32)
    oh1 = (ecol == i1).astype(jnp.float32)
    tri = (lax.broadcasted_iota(jnp.int32, (S, S), 1)
           < lax.broadcasted_iota(jnp.int32, (S, S), 0)).astype(jnp.float32)
    cum = jnp.dot(tri, jnp.concatenate([oh0, oh1], axis=1),
                  preferred_element_type=jnp.float32)
    cum0, cum1 = cum[:, :E], cum[:, E:]
    c0 = jnp.sum(oh0, axis=0, keepdims=True)
    c1 = jnp.sum(oh1, axis=0, keepdims=True)
    tot = c0 + c1
    tot_pad = jnp.ceil(tot / TILE) * TILE
    tri8 = (lax.broadcasted_iota(jnp.int32, (E, E), 0)
            < lax.broadcasted_iota(jnp.int32, (E, E), 1)).astype(jnp.float32)
    start = jnp.dot(tot_pad, tri8, preferred_element_type=jnp.float32)
    pos0 = (jnp.sum(oh0 * start, axis=1, keepdims=True)
            + jnp.sum(oh0 * cum0, axis=1, keepdims=True))
    pos1 = (jnp.sum(oh1 * (start + c0), axis=1, keepdims=True)
            + jnp.sum(oh1 * cum1, axis=1, keepdims=True))
    pos0_ref[...] = pos0.astype(jnp.int32)
    pos1_ref[...] = pos1.astype(jnp.int32)
    jt = (lax.broadcasted_iota(jnp.int32, (NT, E), 0) * TILE).astype(jnp.float32)
    te = jnp.sum((jnp.broadcast_to(start, (NT, E)) <= jt).astype(jnp.int32),
                 axis=1, keepdims=True) - 1
    te_ref[...] = te


def _ffn_body(te_ref, xg_ref, w1_ref, b1_ref, w2_ref, b2_ref, wg_ref, out_ref):
    h = jnp.dot(xg_ref[...], w1_ref[0].T,
                preferred_element_type=jnp.float32) + b1_ref[0]
    h = jax.nn.gelu(h, approximate=True)
    y = jnp.dot(h, w2_ref[0].T,
                preferred_element_type=jnp.float32) + b2_ref[0]
    out_ref[...] = y * wg_ref[...]


def _make_dispatch(S, D, RPAD, TPW):
    mesh = plsc.VectorSubcoreMesh(core_axis_name="c", subcore_axis_name="s",
                                  num_cores=NC, num_subcores=NS)

    @functools.partial(
        pl.kernel,
        out_type=(jax.ShapeDtypeStruct((RPAD, D), jnp.float32),
                  jax.ShapeDtypeStruct((RPAD,), jnp.float32)),
        mesh=mesh,
        scratch_types=(pltpu.VMEM((TPW,), jnp.int32),
                       pltpu.VMEM((TPW,), jnp.int32),
                       pltpu.VMEM((TPW, D), jnp.float32),
                       pltpu.VMEM((TPW,), jnp.float32),
                       pltpu.VMEM((TPW,), jnp.float32),
                       pltpu.SemaphoreType.DMA, pltpu.SemaphoreType.DMA,
                       pltpu.SemaphoreType.DMA, pltpu.SemaphoreType.DMA))
    def dispatch(x_hbm, pos0_hbm, pos1_hbm, w0_hbm, w1_hbm, xg_hbm, wg_hbm,
                 idx0_v, idx1_v, x_v, w0_v, w1_v, s0, s1, s2, s3):
        wid = lax.axis_index("s") * NC + lax.axis_index("c")
        base = wid * TPW
        pltpu.sync_copy(pos0_hbm.at[pl.ds(base, TPW)], idx0_v)
        pltpu.sync_copy(pos1_hbm.at[pl.ds(base, TPW)], idx1_v)
        pltpu.sync_copy(x_hbm.at[pl.ds(base, TPW)], x_v)
        pltpu.sync_copy(w0_hbm.at[pl.ds(base, TPW)], w0_v)
        pltpu.sync_copy(w1_hbm.at[pl.ds(base, TPW)], w1_v)
        c0 = pltpu.async_copy(x_v, xg_hbm.at[idx0_v], s0)
        c1 = pltpu.async_copy(x_v, xg_hbm.at[idx1_v], s1)
        c2 = pltpu.async_copy(w0_v, wg_hbm.at[idx0_v], s2)
        c3 = pltpu.async_copy(w1_v, wg_hbm.at[idx1_v], s3)
        c0.wait()
        c1.wait()
        c2.wait()
        c3.wait()

    return dispatch


def _make_combine(S, D, RPAD, TPW):
    mesh = plsc.VectorSubcoreMesh(core_axis_name="c", subcore_axis_name="s",
                                  num_cores=NC, num_subcores=NS)

    @functools.partial(
        pl.kernel,
        out_type=jax.ShapeDtypeStruct((S, D), jnp.float32),
        mesh=mesh,
        scratch_types=(pltpu.VMEM((TPW,), jnp.int32),
                       pltpu.VMEM((TPW,), jnp.int32),
                       pltpu.VMEM((TPW, D), jnp.float32),
                       pltpu.VMEM((TPW, D), jnp.float32),
                       pltpu.SemaphoreType.DMA, pltpu.SemaphoreType.DMA))
    def combine(yg_hbm, pos0_hbm, pos1_hbm, out_hbm,
                idx0_v, idx1_v, r0_v, r1_v, s0, s1):
        wid = lax.axis_index("s") * NC + lax.axis_index("c")
        base = wid * TPW
        pltpu.sync_copy(pos0_hbm.at[pl.ds(base, TPW)], idx0_v)
        pltpu.sync_copy(pos1_hbm.at[pl.ds(base, TPW)], idx1_v)
        c0 = pltpu.async_copy(yg_hbm.at[idx0_v], r0_v, s0)
        c1 = pltpu.async_copy(yg_hbm.at[idx1_v], r1_v, s1)
        c0.wait()
        c1.wait()

        def row_body(i, carry):
            for j in range(D // 16):
                sl = pl.ds(j * 16, 16)
                r0_v[i, sl] = r0_v[i, sl] + r1_v[i, sl]
            return carry

        lax.fori_loop(0, TPW, row_body, 0)
        pltpu.sync_copy(r0_v, out_hbm.at[pl.ds(base, TPW)])

    return combine


def kernel(x, scale_emb, Wr, br, W1, b1, W2, b2, scale_idx):
    B, S, D = x.shape
    _, H, _ = W1.shape
    NT = (2 * S) // TILE + E
    RPAD = NT * TILE
    TPW = S // NW

    xs = x.reshape(B * S, D)
    semb = lax.dynamic_slice_in_dim(scale_emb, scale_idx, 1, axis=0)

    pos0c, pos1c, w0c, w1c, tec = pl.pallas_call(
        _router_body,
        in_specs=[
            pl.BlockSpec((B * S, D), lambda: (0, 0)),
            pl.BlockSpec((1, D), lambda: (0, 0)),
            pl.BlockSpec((E, D), lambda: (0, 0)),
            pl.BlockSpec((1, E), lambda: (0, 0)),
        ],
        out_specs=[
            pl.BlockSpec((B * S, 1), lambda: (0, 0)),
            pl.BlockSpec((B * S, 1), lambda: (0, 0)),
            pl.BlockSpec((B * S, 1), lambda: (0, 0)),
            pl.BlockSpec((B * S, 1), lambda: (0, 0)),
            pl.BlockSpec((NT, 1), lambda: (0, 0)),
        ],
        out_shape=[
            jax.ShapeDtypeStruct((B * S, 1), jnp.int32),
            jax.ShapeDtypeStruct((B * S, 1), jnp.int32),
            jax.ShapeDtypeStruct((B * S, 1), jnp.float32),
            jax.ShapeDtypeStruct((B * S, 1), jnp.float32),
            jax.ShapeDtypeStruct((NT, 1), jnp.int32),
        ],
    )(xs, semb, Wr, br.reshape(1, E))
    pos0 = pos0c.reshape(B * S)
    pos1 = pos1c.reshape(B * S)
    te = tec.reshape(NT)

    xg, wg = _make_dispatch(B * S, D, RPAD, TPW)(
        xs, pos0, pos1, w0c.reshape(B * S), w1c.reshape(B * S))

    yg = pl.pallas_call(
        _ffn_body,
        grid_spec=pltpu.PrefetchScalarGridSpec(
            num_scalar_prefetch=1,
            grid=(NT,),
            in_specs=[
                pl.BlockSpec((TILE, D), lambda i, te: (i, 0)),
                pl.BlockSpec((1, H, D), lambda i, te: (te[i], 0, 0)),
                pl.BlockSpec((1, 1, H), lambda i, te: (te[i], 0, 0)),
                pl.BlockSpec((1, D, H), lambda i, te: (te[i], 0, 0)),
                pl.BlockSpec((1, 1, D), lambda i, te: (te[i], 0, 0)),
                pl.BlockSpec((TILE, 1), lambda i, te: (i, 0)),
            ],
            out_specs=pl.BlockSpec((TILE, D), lambda i, te: (i, 0)),
        ),
        out_shape=jax.ShapeDtypeStruct((RPAD, D), jnp.float32),
    )(te, xg, W1, b1.reshape(E, 1, H), W2, b2.reshape(E, 1, D),
      wg.reshape(RPAD, 1))

    out = _make_combine(B * S, D, RPAD, TPW)(yg, pos0, pos1)
    return out.reshape(B, S, D)


# speedup vs baseline: 1.4409x; 1.4409x over previous
import functools
import jax
import jax.numpy as jnp
from jax import lax
from jax.experimental import pallas as pl
from jax.experimental.pallas import tpu as pltpu
from jax.experimental.pallas import tpu_sc as plsc

E = 8
TILE = 256
NC, NS = 2, 16
NW = NC * NS


def _router_body(x_ref, semb_ref, wr_ref, br_ref,
                 pos0_ref, pos1_ref, w0_ref, w1_ref, te_ref):
    S = x_ref.shape[0]
    NT = te_ref.shape[0]
    logits = jnp.dot(x_ref[...] + semb_ref[...], wr_ref[...].T,
                     preferred_element_type=jnp.float32) + br_ref[...]
    ecol = lax.broadcasted_iota(jnp.int32, (S, E), 1)
    big = jnp.int32(E)
    l0 = jnp.max(logits, axis=-1, keepdims=True)
    i0 = jnp.min(jnp.where(logits == l0, ecol, big), axis=-1, keepdims=True)
    masked = jnp.where(ecol == i0, -jnp.inf, logits)
    l1 = jnp.max(masked, axis=-1, keepdims=True)
    i1 = jnp.min(jnp.where(masked == l1, ecol, big), axis=-1, keepdims=True)
    w0 = 1.0 / (1.0 + jnp.exp(l1 - l0))
    w0_ref[...] = jnp.broadcast_to(w0, w0_ref.shape)
    w1_ref[...] = jnp.broadcast_to(1.0 - w0, w1_ref.shape)

    oh0 = (ecol == i0).astype(jnp.float32)
    oh1 = (ecol == i1).astype(jnp.float32)
    tri = (lax.broadcasted_iota(jnp.int32, (S, S), 1)
           < lax.broadcasted_iota(jnp.int32, (S, S), 0)).astype(jnp.float32)
    cum = jnp.dot(tri, jnp.concatenate([oh0, oh1], axis=1),
                  preferred_element_type=jnp.float32)
    cum0, cum1 = cum[:, :E], cum[:, E:]
    c0 = jnp.sum(oh0, axis=0, keepdims=True)
    c1 = jnp.sum(oh1, axis=0, keepdims=True)
    tot = c0 + c1
    tot_pad = jnp.ceil(tot / TILE) * TILE
    tri8 = (lax.broadcasted_iota(jnp.int32, (E, E), 0)
            < lax.broadcasted_iota(jnp.int32, (E, E), 1)).astype(jnp.float32)
    start = jnp.dot(tot_pad, tri8, preferred_element_type=jnp.float32)
    pos0 = (jnp.sum(oh0 * start, axis=1, keepdims=True)
            + jnp.sum(oh0 * cum0, axis=1, keepdims=True))
    pos1 = (jnp.sum(oh1 * (start + c0), axis=1, keepdims=True)
            + jnp.sum(oh1 * cum1, axis=1, keepdims=True))
    pos0_ref[...] = pos0.astype(jnp.int32)
    pos1_ref[...] = pos1.astype(jnp.int32)
    jt = (lax.broadcasted_iota(jnp.int32, (NT, E), 0) * TILE).astype(jnp.float32)
    te = jnp.sum((jnp.broadcast_to(start, (NT, E)) <= jt).astype(jnp.int32),
                 axis=1, keepdims=True) - 1
    te_ref[...] = te


def _ffn_body(te_ref, xg_ref, wg_ref, w1_hbm, b1_ref, w2_hbm, b2_ref, out_ref,
              w1_v, w2_v, s1, s2):
    i = pl.program_id(0)
    nt = pl.num_programs(0)
    e = te_ref[i]
    slot = lax.rem(e, 2)

    @pl.when(i == 0)
    def _():
        pltpu.make_async_copy(w1_hbm.at[e], w1_v.at[slot], s1.at[slot]).start()
        pltpu.make_async_copy(w2_hbm.at[e], w2_v.at[slot], s2.at[slot]).start()

    changed = jnp.logical_or(i == 0, te_ref[jnp.maximum(i - 1, 0)] != e)

    @pl.when(changed)
    def _():
        pltpu.make_async_copy(w1_hbm.at[e], w1_v.at[slot], s1.at[slot]).wait()
        pltpu.make_async_copy(w2_hbm.at[e], w2_v.at[slot], s2.at[slot]).wait()

    en = te_ref[jnp.minimum(i + 1, nt - 1)]

    @pl.when(en != e)
    def _():
        sn = lax.rem(en, 2)
        pltpu.make_async_copy(w1_hbm.at[en], w1_v.at[sn], s1.at[sn]).start()
        pltpu.make_async_copy(w2_hbm.at[en], w2_v.at[sn], s2.at[sn]).start()

    h = jnp.dot(xg_ref[...], w1_v[slot].T,
                preferred_element_type=jnp.float32) + b1_ref[e]
    h = jax.nn.gelu(h, approximate=True)
    y = jnp.dot(h, w2_v[slot].T, preferred_element_type=jnp.float32) + b2_ref[e]
    out_ref[...] = y * wg_ref[:, 0:1]


def _make_dispatch(S, D, RPAD, TPW):
    mesh = plsc.VectorSubcoreMesh(core_axis_name="c", subcore_axis_name="s",
                                  num_cores=NC, num_subcores=NS)

    @functools.partial(
        pl.kernel,
        out_type=(jax.ShapeDtypeStruct((RPAD, D), jnp.float32),
                  jax.ShapeDtypeStruct((RPAD, 128), jnp.float32)),
        mesh=mesh,
        scratch_types=(pltpu.VMEM((TPW,), jnp.int32),
                       pltpu.VMEM((TPW,), jnp.int32),
                       pltpu.VMEM((TPW, D), jnp.float32),
                       pltpu.VMEM((TPW, 128), jnp.float32),
                       pltpu.VMEM((TPW, 128), jnp.float32),
                       pltpu.SemaphoreType.DMA, pltpu.SemaphoreType.DMA,
                       pltpu.SemaphoreType.DMA, pltpu.SemaphoreType.DMA))
    def dispatch(x_hbm, pos0_hbm, pos1_hbm, w0_hbm, w1_hbm, xg_hbm, wg_hbm,
                 idx0_v, idx1_v, x_v, w0_v, w1_v, s0, s1, s2, s3):
        wid = lax.axis_index("s") * NC + lax.axis_index("c")
        base = wid * TPW
        pltpu.sync_copy(pos0_hbm.at[pl.ds(base, TPW)], idx0_v)
        pltpu.sync_copy(pos1_hbm.at[pl.ds(base, TPW)], idx1_v)
        pltpu.sync_copy(x_hbm.at[pl.ds(base, TPW)], x_v)
        pltpu.sync_copy(w0_hbm.at[pl.ds(base, TPW)], w0_v)
        pltpu.sync_copy(w1_hbm.at[pl.ds(base, TPW)], w1_v)
        c0 = pltpu.async_copy(x_v, xg_hbm.at[idx0_v], s0)
        c1 = pltpu.async_copy(x_v, xg_hbm.at[idx1_v], s1)
        c2 = pltpu.async_copy(w0_v, wg_hbm.at[idx0_v], s2)
        c3 = pltpu.async_copy(w1_v, wg_hbm.at[idx1_v], s3)
        c0.wait()
        c1.wait()
        c2.wait()
        c3.wait()

    return dispatch


def _make_combine(S, D, RPAD, TPW):
    mesh = plsc.VectorSubcoreMesh(core_axis_name="c", subcore_axis_name="s",
                                  num_cores=NC, num_subcores=NS)

    @functools.partial(
        pl.kernel,
        out_type=jax.ShapeDtypeStruct((S, D), jnp.float32),
        mesh=mesh,
        scratch_types=(pltpu.VMEM((TPW,), jnp.int32),
                       pltpu.VMEM((TPW,), jnp.int32),
                       pltpu.VMEM((TPW, D), jnp.float32),
                       pltpu.VMEM((TPW, D), jnp.float32),
                       pltpu.SemaphoreType.DMA, pltpu.SemaphoreType.DMA))
    def combine(yg_hbm, pos0_hbm, pos1_hbm, out_hbm,
                idx0_v, idx1_v, r0_v, r1_v, s0, s1):
        wid = lax.axis_index("s") * NC + lax.axis_index("c")
        base = wid * TPW
        pltpu.sync_copy(pos0_hbm.at[pl.ds(base, TPW)], idx0_v)
        pltpu.sync_copy(pos1_hbm.at[pl.ds(base, TPW)], idx1_v)
        c0 = pltpu.async_copy(yg_hbm.at[idx0_v], r0_v, s0)
        c1 = pltpu.async_copy(yg_hbm.at[idx1_v], r1_v, s1)
        c0.wait()
        c1.wait()

        def row_body(i, carry):
            for j in range(D // 16):
                sl = pl.ds(j * 16, 16)
                r0_v[i, sl] = r0_v[i, sl] + r1_v[i, sl]
            return carry

        lax.fori_loop(0, TPW, row_body, 0)
        pltpu.sync_copy(r0_v, out_hbm.at[pl.ds(base, TPW)])

    return combine


def kernel(x, scale_emb, Wr, br, W1, b1, W2, b2, scale_idx):
    B, S, D = x.shape
    _, H, _ = W1.shape
    NT = (2 * S) // TILE + E
    RPAD = NT * TILE
    TPW = S // NW

    xs = x.reshape(B * S, D)
    semb = lax.dynamic_slice_in_dim(scale_emb, scale_idx, 1, axis=0)

    pos0c, pos1c, w0c, w1c, tec = pl.pallas_call(
        _router_body,
        in_specs=[
            pl.BlockSpec((B * S, D), lambda: (0, 0)),
            pl.BlockSpec((1, D), lambda: (0, 0)),
            pl.BlockSpec((E, D), lambda: (0, 0)),
            pl.BlockSpec((1, E), lambda: (0, 0)),
        ],
        out_specs=[
            pl.BlockSpec((B * S, 1), lambda: (0, 0)),
            pl.BlockSpec((B * S, 1), lambda: (0, 0)),
            pl.BlockSpec((B * S, 128), lambda: (0, 0)),
            pl.BlockSpec((B * S, 128), lambda: (0, 0)),
            pl.BlockSpec((NT, 1), lambda: (0, 0)),
        ],
        out_shape=[
            jax.ShapeDtypeStruct((B * S, 1), jnp.int32),
            jax.ShapeDtypeStruct((B * S, 1), jnp.int32),
            jax.ShapeDtypeStruct((B * S, 128), jnp.float32),
            jax.ShapeDtypeStruct((B * S, 128), jnp.float32),
            jax.ShapeDtypeStruct((NT, 1), jnp.int32),
        ],
    )(xs, semb, Wr, br.reshape(1, E))
    pos0 = pos0c.reshape(B * S)
    pos1 = pos1c.reshape(B * S)
    te = tec.reshape(NT)

    xg, wg = _make_dispatch(B * S, D, RPAD, TPW)(xs, pos0, pos1, w0c, w1c)

    yg = pl.pallas_call(
        _ffn_body,
        grid_spec=pltpu.PrefetchScalarGridSpec(
            num_scalar_prefetch=1,
            grid=(NT,),
            in_specs=[
                pl.BlockSpec((TILE, D), lambda i, te: (i, 0)),
                pl.BlockSpec((TILE, 128), lambda i, te: (i, 0)),
                pl.BlockSpec(memory_space=pl.ANY),
                pl.BlockSpec((E, 1, H), lambda i, te: (0, 0, 0)),
                pl.BlockSpec(memory_space=pl.ANY),
                pl.BlockSpec((E, 1, D), lambda i, te: (0, 0, 0)),
            ],
            out_specs=pl.BlockSpec((TILE, D), lambda i, te: (i, 0)),
            scratch_shapes=[
                pltpu.VMEM((2, H, D), jnp.float32),
                pltpu.VMEM((2, D, H), jnp.float32),
                pltpu.SemaphoreType.DMA((2,)),
                pltpu.SemaphoreType.DMA((2,)),
            ],
        ),
        out_shape=jax.ShapeDtypeStruct((RPAD, D), jnp.float32),
    )(te, xg, wg, W1, b1.reshape(E, 1, H), W2, b2.reshape(E, 1, D))

    out = _make_combine(B * S, D, RPAD, TPW)(yg, pos0, pos1)
    return out.reshape(B, S, D)


# speedup vs baseline: 1.5113x; 1.0489x over previous
import functools
import jax
import jax.numpy as jnp
from jax import lax
from jax.experimental import pallas as pl
from jax.experimental.pallas import tpu as pltpu
from jax.experimental.pallas import tpu_sc as plsc

E = 8
TILE = 256
NC, NS = 2, 16
NW = NC * NS


def _router_body(x_ref, semb_ref, wr_ref, br_ref,
                 pos0_ref, pos1_ref, w0_ref, w1_ref, te_ref):
    S = x_ref.shape[0]
    NT = te_ref.shape[0]
    logits = jnp.dot(x_ref[...] + semb_ref[...], wr_ref[...].T,
                     preferred_element_type=jnp.float32) + br_ref[...]
    ecol = lax.broadcasted_iota(jnp.int32, (S, E), 1)
    big = jnp.int32(E)
    l0 = jnp.max(logits, axis=-1, keepdims=True)
    i0 = jnp.min(jnp.where(logits == l0, ecol, big), axis=-1, keepdims=True)
    masked = jnp.where(ecol == i0, -jnp.inf, logits)
    l1 = jnp.max(masked, axis=-1, keepdims=True)
    i1 = jnp.min(jnp.where(masked == l1, ecol, big), axis=-1, keepdims=True)
    w0 = 1.0 / (1.0 + jnp.exp(l1 - l0))
    w0_ref[...] = jnp.broadcast_to(w0, w0_ref.shape)
    w1_ref[...] = jnp.broadcast_to(1.0 - w0, w1_ref.shape)

    oh0 = (ecol == i0).astype(jnp.float32)
    oh1 = (ecol == i1).astype(jnp.float32)
    C = 256 if S % 256 == 0 else S
    G = S // C
    oh2 = jnp.concatenate([oh0, oh1], axis=1)
    triC = (lax.broadcasted_iota(jnp.int32, (C, C), 1)
            < lax.broadcasted_iota(jnp.int32, (C, C), 0)).astype(jnp.float32)
    parts = []
    off = jnp.zeros((1, 2 * E), jnp.float32)
    for g in range(G):
        blk = lax.slice_in_dim(oh2, g * C, (g + 1) * C, axis=0)
        lcum = jnp.dot(triC, blk, preferred_element_type=jnp.float32)
        parts.append(lcum + off)
        off = off + jnp.sum(blk, axis=0, keepdims=True)
    cum = jnp.concatenate(parts, axis=0)
    cum0, cum1 = cum[:, :E], cum[:, E:]
    c0, c1 = off[:, :E], off[:, E:]
    tot = c0 + c1
    tot_pad = jnp.ceil(tot / TILE) * TILE
    tri8 = (lax.broadcasted_iota(jnp.int32, (E, E), 0)
            < lax.broadcasted_iota(jnp.int32, (E, E), 1)).astype(jnp.float32)
    start = jnp.dot(tot_pad, tri8, preferred_element_type=jnp.float32)
    pos0 = (jnp.sum(oh0 * start, axis=1, keepdims=True)
            + jnp.sum(oh0 * cum0, axis=1, keepdims=True))
    pos1 = (jnp.sum(oh1 * (start + c0), axis=1, keepdims=True)
            + jnp.sum(oh1 * cum1, axis=1, keepdims=True))
    pos0_ref[...] = pos0.astype(jnp.int32)
    pos1_ref[...] = pos1.astype(jnp.int32)
    jt = (lax.broadcasted_iota(jnp.int32, (NT, E), 0) * TILE).astype(jnp.float32)
    te = jnp.sum((jnp.broadcast_to(start, (NT, E)) <= jt).astype(jnp.int32),
                 axis=1, keepdims=True) - 1
    te_ref[...] = te


def _ffn_body(te_ref, xg_ref, wg_ref, w1_hbm, b1_ref, w2_hbm, b2_ref, out_ref,
              w1_v, w2_v, s1, s2):
    i = pl.program_id(0)
    nt = pl.num_programs(0)
    e = te_ref[i]
    slot = lax.rem(e, 2)

    @pl.when(i == 0)
    def _():
        pltpu.make_async_copy(w1_hbm.at[e], w1_v.at[slot], s1.at[slot]).start()
        pltpu.make_async_copy(w2_hbm.at[e], w2_v.at[slot], s2.at[slot]).start()

    changed = jnp.logical_or(i == 0, te_ref[jnp.maximum(i - 1, 0)] != e)

    @pl.when(changed)
    def _():
        pltpu.make_async_copy(w1_hbm.at[e], w1_v.at[slot], s1.at[slot]).wait()
        pltpu.make_async_copy(w2_hbm.at[e], w2_v.at[slot], s2.at[slot]).wait()

    en = te_ref[jnp.minimum(i + 1, nt - 1)]

    @pl.when(en != e)
    def _():
        sn = lax.rem(en, 2)
        pltpu.make_async_copy(w1_hbm.at[en], w1_v.at[sn], s1.at[sn]).start()
        pltpu.make_async_copy(w2_hbm.at[en], w2_v.at[sn], s2.at[sn]).start()

    h = jnp.dot(xg_ref[...], w1_v[slot].T,
                preferred_element_type=jnp.float32) + b1_ref[e]
    h = jax.nn.gelu(h, approximate=True)
    y = jnp.dot(h, w2_v[slot].T, preferred_element_type=jnp.float32) + b2_ref[e]
    out_ref[...] = y * wg_ref[:, 0:1]


def _make_dispatch(S, D, RPAD, TPW):
    mesh = plsc.VectorSubcoreMesh(core_axis_name="c", subcore_axis_name="s",
                                  num_cores=NC, num_subcores=NS)

    @functools.partial(
        pl.kernel,
        out_type=(jax.ShapeDtypeStruct((RPAD, D), jnp.float32),
                  jax.ShapeDtypeStruct((RPAD, 128), jnp.float32)),
        mesh=mesh,
        scratch_types=(pltpu.VMEM((TPW,), jnp.int32),
                       pltpu.VMEM((TPW,), jnp.int32),
                       pltpu.VMEM((TPW, D), jnp.float32),
                       pltpu.VMEM((TPW, 128), jnp.float32),
                       pltpu.VMEM((TPW, 128), jnp.float32),
                       pltpu.SemaphoreType.DMA, pltpu.SemaphoreType.DMA,
                       pltpu.SemaphoreType.DMA, pltpu.SemaphoreType.DMA))
    def dispatch(x_hbm, pos0_hbm, pos1_hbm, w0_hbm, w1_hbm, xg_hbm, wg_hbm,
                 idx0_v, idx1_v, x_v, w0_v, w1_v, s0, s1, s2, s3):
        wid = lax.axis_index("s") * NC + lax.axis_index("c")
        base = wid * TPW
        pltpu.sync_copy(pos0_hbm.at[pl.ds(base, TPW)], idx0_v)
        pltpu.sync_copy(pos1_hbm.at[pl.ds(base, TPW)], idx1_v)
        pltpu.sync_copy(x_hbm.at[pl.ds(base, TPW)], x_v)
        pltpu.sync_copy(w0_hbm.at[pl.ds(base, TPW)], w0_v)
        pltpu.sync_copy(w1_hbm.at[pl.ds(base, TPW)], w1_v)
        c0 = pltpu.async_copy(x_v, xg_hbm.at[idx0_v], s0)
        c1 = pltpu.async_copy(x_v, xg_hbm.at[idx1_v], s1)
        c2 = pltpu.async_copy(w0_v, wg_hbm.at[idx0_v], s2)
        c3 = pltpu.async_copy(w1_v, wg_hbm.at[idx1_v], s3)
        c0.wait()
        c1.wait()
        c2.wait()
        c3.wait()

    return dispatch


def _make_combine(S, D, RPAD, TPW):
    mesh = plsc.VectorSubcoreMesh(core_axis_name="c", subcore_axis_name="s",
                                  num_cores=NC, num_subcores=NS)

    @functools.partial(
        pl.kernel,
        out_type=jax.ShapeDtypeStruct((S, D), jnp.float32),
        mesh=mesh,
        scratch_types=(pltpu.VMEM((TPW,), jnp.int32),
                       pltpu.VMEM((TPW,), jnp.int32),
                       pltpu.VMEM((TPW, D), jnp.float32),
                       pltpu.VMEM((TPW, D), jnp.float32),
                       pltpu.SemaphoreType.DMA, pltpu.SemaphoreType.DMA))
    def combine(yg_hbm, pos0_hbm, pos1_hbm, out_hbm,
                idx0_v, idx1_v, r0_v, r1_v, s0, s1):
        wid = lax.axis_index("s") * NC + lax.axis_index("c")
        base = wid * TPW
        pltpu.sync_copy(pos0_hbm.at[pl.ds(base, TPW)], idx0_v)
        pltpu.sync_copy(pos1_hbm.at[pl.ds(base, TPW)], idx1_v)
        c0 = pltpu.async_copy(yg_hbm.at[idx0_v], r0_v, s0)
        c1 = pltpu.async_copy(yg_hbm.at[idx1_v], r1_v, s1)
        c0.wait()
        c1.wait()

        def row_body(i, carry):
            for j in range(D // 16):
                sl = pl.ds(j * 16, 16)
                r0_v[i, sl] = r0_v[i, sl] + r1_v[i, sl]
            return carry

        lax.fori_loop(0, TPW, row_body, 0)
        pltpu.sync_copy(r0_v, out_hbm.at[pl.ds(base, TPW)])

    return combine


def kernel(x, scale_emb, Wr, br, W1, b1, W2, b2, scale_idx):
    B, S, D = x.shape
    _, H, _ = W1.shape
    NT = (2 * S) // TILE + E
    RPAD = NT * TILE
    TPW = S // NW

    xs = x.reshape(B * S, D)
    semb = lax.dynamic_slice_in_dim(scale_emb, scale_idx, 1, axis=0)

    pos0c, pos1c, w0c, w1c, tec = pl.pallas_call(
        _router_body,
        in_specs=[
            pl.BlockSpec((B * S, D), lambda: (0, 0)),
            pl.BlockSpec((1, D), lambda: (0, 0)),
            pl.BlockSpec((E, D), lambda: (0, 0)),
            pl.BlockSpec((1, E), lambda: (0, 0)),
        ],
        out_specs=[
            pl.BlockSpec((B * S, 1), lambda: (0, 0)),
            pl.BlockSpec((B * S, 1), lambda: (0, 0)),
            pl.BlockSpec((B * S, 128), lambda: (0, 0)),
            pl.BlockSpec((B * S, 128), lambda: (0, 0)),
            pl.BlockSpec((NT, 1), lambda: (0, 0)),
        ],
        out_shape=[
            jax.ShapeDtypeStruct((B * S, 1), jnp.int32),
            jax.ShapeDtypeStruct((B * S, 1), jnp.int32),
            jax.ShapeDtypeStruct((B * S, 128), jnp.float32),
            jax.ShapeDtypeStruct((B * S, 128), jnp.float32),
            jax.ShapeDtypeStruct((NT, 1), jnp.int32),
        ],
    )(xs, semb, Wr, br.reshape(1, E))
    pos0 = pos0c.reshape(B * S)
    pos1 = pos1c.reshape(B * S)
    te = tec.reshape(NT)

    xg, wg = _make_dispatch(B * S, D, RPAD, TPW)(xs, pos0, pos1, w0c, w1c)

    yg = pl.pallas_call(
        _ffn_body,
        grid_spec=pltpu.PrefetchScalarGridSpec(
            num_scalar_prefetch=1,
            grid=(NT,),
            in_specs=[
                pl.BlockSpec((TILE, D), lambda i, te: (i, 0)),
                pl.BlockSpec((TILE, 128), lambda i, te: (i, 0)),
                pl.BlockSpec(memory_space=pl.ANY),
                pl.BlockSpec((E, 1, H), lambda i, te: (0, 0, 0)),
                pl.BlockSpec(memory_space=pl.ANY),
                pl.BlockSpec((E, 1, D), lambda i, te: (0, 0, 0)),
            ],
            out_specs=pl.BlockSpec((TILE, D), lambda i, te: (i, 0)),
            scratch_shapes=[
                pltpu.VMEM((2, H, D), jnp.float32),
                pltpu.VMEM((2, D, H), jnp.float32),
                pltpu.SemaphoreType.DMA((2,)),
                pltpu.SemaphoreType.DMA((2,)),
            ],
        ),
        out_shape=jax.ShapeDtypeStruct((RPAD, D), jnp.float32),
    )(te, xg, wg, W1, b1.reshape(E, 1, H), W2, b2.reshape(E, 1, D))

    out = _make_combine(B * S, D, RPAD, TPW)(yg, pos0, pos1)
    return out.reshape(B, S, D)
